# baseline (speedup 1.0000x reference)
import functools

import jax
import jax.numpy as jnp
from jax import lax
from jax.experimental import pallas as pl
from jax.experimental.pallas import tpu as pltpu
from jax.experimental.pallas import tpu_sc as plsc

NUM_EXPERTS = 64
TOP_K = 2
HIDDEN = 2048
T = 16384

ROWS = 2048

_info = plsc.get_sparse_core_info()
NC = _info.num_cores
NS = _info.num_subcores
L = _info.num_lanes
NW = NC * NS
TPW = T // NW
GROUPS = TPW // L


def _softmax_body(h_ref, w_ref, probs_ref):
    h = h_ref[...]
    w = w_ref[...]
    logits = jax.lax.dot_general(
        h, w,
        dimension_numbers=(((1,), (1,)), ((), ())),
        preferred_element_type=jnp.float32,
    )
    m = jnp.max(logits, axis=-1, keepdims=True)
    e = jnp.exp(logits - m)
    s = jnp.sum(e, axis=-1, keepdims=True)
    probs_ref[...] = e / s


def _tc_softmax(hidden_states, weight):
    return pl.pallas_call(
        _softmax_body,
        grid=(T // ROWS,),
        in_specs=[
            pl.BlockSpec((ROWS, HIDDEN), lambda i: (i, 0)),
            pl.BlockSpec((NUM_EXPERTS, HIDDEN), lambda i: (0, 0)),
        ],
        out_specs=pl.BlockSpec((ROWS, NUM_EXPERTS), lambda i: (i, 0)),
        out_shape=jax.ShapeDtypeStruct((T, NUM_EXPERTS), jnp.float32),
    )(hidden_states, weight)


@functools.partial(
    pl.kernel,
    out_type=[
        jax.ShapeDtypeStruct((T * TOP_K,), jnp.float32),
        jax.ShapeDtypeStruct((T * TOP_K,), jnp.int32),
    ],
    mesh=plsc.VectorSubcoreMesh(core_axis_name="c", subcore_axis_name="s"),
    compiler_params=pltpu.CompilerParams(needs_layout_passes=False),
    scratch_types=[
        pltpu.VMEM((TPW * NUM_EXPERTS,), jnp.float32),
        pltpu.VMEM((TPW * TOP_K,), jnp.float32),
        pltpu.VMEM((TPW * TOP_K,), jnp.int32),
    ],
)
def _sc_top2(probs_hbm, tw_hbm, ti_hbm, probs_v, tw_v, ti_v):
    wid = lax.axis_index("s") * NC + lax.axis_index("c")
    base = wid * TPW
    pltpu.sync_copy(probs_hbm.at[pl.ds(base * NUM_EXPERTS, TPW * NUM_EXPERTS)],
                    probs_v)

    def group(g, carry):
        rows = g * L + lax.broadcasted_iota(jnp.int32, (L,), 0)
        row_base = rows * NUM_EXPERTS
        top1v = jnp.full((L,), -1.0, jnp.float32)
        top2v = jnp.full((L,), -1.0, jnp.float32)
        top1i = jnp.zeros((L,), jnp.int32)
        top2i = jnp.zeros((L,), jnp.int32)
        for e in range(NUM_EXPERTS):
            col = jnp.full((L,), e, jnp.int32)
            v = plsc.load_gather(probs_v, [row_base + e])
            gt1 = v > top1v
            gt2 = v > top2v
            top2v = jnp.where(gt1, top1v, jnp.where(gt2, v, top2v))
            top2i = jnp.where(gt1, top1i, jnp.where(gt2, col, top2i))
            top1v = jnp.where(gt1, v, top1v)
            top1i = jnp.where(gt1, col, top1i)
        denom = top1v + top2v + 1e-9
        out_base = rows * TOP_K
        plsc.store_scatter(tw_v, [out_base], top1v / denom)
        plsc.store_scatter(tw_v, [out_base + 1], top2v / denom)
        plsc.store_scatter(ti_v, [out_base], top1i)
        plsc.store_scatter(ti_v, [out_base + 1], top2i)
        return carry

    lax.fori_loop(0, GROUPS, group, 0)
    pltpu.sync_copy(tw_v, tw_hbm.at[pl.ds(base * TOP_K, TPW * TOP_K)])
    pltpu.sync_copy(ti_v, ti_hbm.at[pl.ds(base * TOP_K, TPW * TOP_K)])


@jax.jit
def _router(hidden_states, weight):
    probs = _tc_softmax(hidden_states, weight)
    top_w, top_i = _sc_top2(probs.reshape(T * NUM_EXPERTS))
    return probs, top_w.reshape(T, TOP_K), top_i.reshape(T, TOP_K)


def kernel(hidden_states, weight):
    probs, top_w, top_i = _router(hidden_states, weight)
    return probs, top_w.astype(hidden_states.dtype), top_i.astype(jnp.int64)


# speedup vs baseline: 1.7004x; 1.7004x over previous
import functools

import jax
import jax.numpy as jnp
from jax.experimental import pallas as pl
from jax.experimental.pallas import tpu as pltpu

NUM_EXPERTS = 64
TOP_K = 2
HIDDEN = 2048
T = 16384

ROWS = 2048
HALF = HIDDEN // 2


def _router_body(h1_ref, h2_ref, w1_ref, w2_ref, probs_ref, tw_ref, ti_ref):
    logits = jax.lax.dot_general(
        h1_ref[...], w1_ref[...],
        dimension_numbers=(((1,), (1,)), ((), ())),
        preferred_element_type=jnp.float32,
    ) + jax.lax.dot_general(
        h2_ref[...], w2_ref[...],
        dimension_numbers=(((1,), (1,)), ((), ())),
        preferred_element_type=jnp.float32,
    )
    m = jnp.max(logits, axis=-1, keepdims=True)
    e = jnp.exp(logits - m)
    s = jnp.sum(e, axis=-1, keepdims=True)
    probs = e / s
    probs_ref[...] = probs

    lane = jax.lax.broadcasted_iota(jnp.int32, probs.shape, 1)
    m1 = jnp.max(probs, axis=-1, keepdims=True)
    i1 = jnp.min(jnp.where(probs == m1, lane, NUM_EXPERTS), axis=-1, keepdims=True)
    masked = jnp.where(lane == i1, -1.0, probs)
    m2 = jnp.max(masked, axis=-1, keepdims=True)
    i2 = jnp.min(jnp.where(masked == m2, lane, NUM_EXPERTS), axis=-1, keepdims=True)

    denom = m1 + m2 + 1e-9
    tw_ref[...] = jnp.concatenate([m1 / denom, m2 / denom], axis=-1)
    ti_ref[...] = jnp.concatenate([i1, i2], axis=-1)


@jax.jit
def _router(hidden_states, weight):
    grid = (T // ROWS,)
    return pl.pallas_call(
        _router_body,
        grid=grid,
        in_specs=[
            pl.BlockSpec((ROWS, HALF), lambda i: (i, 0)),
            pl.BlockSpec((ROWS, HALF), lambda i: (i, 1)),
            pl.BlockSpec((NUM_EXPERTS, HALF), lambda i: (0, 0)),
            pl.BlockSpec((NUM_EXPERTS, HALF), lambda i: (0, 1)),
        ],
        out_specs=[
            pl.BlockSpec((ROWS, NUM_EXPERTS), lambda i: (i, 0)),
            pl.BlockSpec((ROWS, TOP_K), lambda i: (i, 0)),
            pl.BlockSpec((ROWS, TOP_K), lambda i: (i, 0)),
        ],
        out_shape=[
            jax.ShapeDtypeStruct((T, NUM_EXPERTS), jnp.float32),
            jax.ShapeDtypeStruct((T, TOP_K), jnp.float32),
            jax.ShapeDtypeStruct((T, TOP_K), jnp.int32),
        ],
    )(hidden_states, hidden_states, weight, weight)


def kernel(hidden_states, weight):
    probs, top_w, top_i = _router(hidden_states, weight)
    return probs, top_w.astype(hidden_states.dtype), top_i.astype(jnp.int64)
